# baseline (speedup 1.0000x reference)
import jax
import jax.numpy as jnp
from jax import lax
from jax.experimental import pallas as pl
from jax.experimental.pallas import tpu as pltpu
from jax.experimental.pallas import tpu_sc as plsc

_H = 16
_S = 4096
_NC = 2
_NS = 16
_NW = _NC * _NS
_ROWS_PER_W = _H * _S // _NW
_BLOCKS_PER_W = _ROWS_PER_W // 8
_LAG = 4


def _sc_body(shifts_hbm, out_hbm, buf, sem):
    wid = lax.axis_index("s") * _NC + lax.axis_index("c")
    h = wid // 2
    half = wid % 2
    pltpu.sync_copy(shifts_hbm.at[h], buf)

    row_base = h * _S + half * _ROWS_PER_W
    q_base = 511 - (_BLOCKS_PER_W * half)

    drain_one = pltpu.make_async_copy(
        buf.at[:, pl.ds(0, _S)], out_hbm.at[pl.ds(0, 8), :], sem)

    def body(b, _):
        q = q_base - b
        src = buf.at[:, pl.ds(pl.multiple_of(8 * q, 8), _S)]
        dst = out_hbm.at[pl.ds(pl.multiple_of(row_base + 8 * b, 8), 8), :]
        pltpu.async_copy(src, dst, sem)

        @pl.when(b >= _LAG)
        def _():
            drain_one.wait()
        return 0

    lax.fori_loop(0, _BLOCKS_PER_W, body, 0)
    for _ in range(_LAG):
        drain_one.wait()


@jax.jit
def _expand(shifts):
    mesh = plsc.VectorSubcoreMesh(core_axis_name="c", subcore_axis_name="s")
    return pl.kernel(
        _sc_body,
        out_type=jax.ShapeDtypeStruct((_H * _S, _S), jnp.float32),
        mesh=mesh,
        scratch_types=[
            pltpu.VMEM((8, 8192), jnp.float32),
            pltpu.SemaphoreType.DMA,
        ],
        compiler_params=pltpu.CompilerParams(use_tc_tiling_on_sc=False),
    )(shifts)


def kernel(qlen, klen, relative_attention_bias):
    tt = relative_attention_bias.T
    ttp = jnp.pad(tt, ((0, 0), (0, 8192 + 7 - tt.shape[1])))
    shifts = jnp.stack([ttp[:, 7 - s:7 - s + 8192] for s in range(8)], axis=1)
    out = _expand(shifts)
    return out.reshape(1, _H, _S, _S)
